# baseline (speedup 1.0000x reference)
import functools

import jax
import jax.numpy as jnp
from jax import lax
from jax.experimental import pallas as pl
from jax.experimental.pallas import tpu as pltpu
from jax.experimental.pallas import tpu_sc as plsc

_CH = 16
_K = 4
_NBUF = 2


def _sc_gather_kernel(nc, bpw, nchunk, table_ref, tflat_ref, idx_ref,
                      pick_ref, out_ref, part_ref, idx_v, pick_v, picked_v,
                      rows_v, part_v, gsems, psem):
    w = lax.axis_index("s") * nc + lax.axis_index("c")
    base = w * bpw
    pltpu.sync_copy(idx_ref.at[w], idx_v)
    pltpu.sync_copy(pick_ref.at[w], pick_v)
    pltpu.make_async_copy(tflat_ref.at[pick_v], picked_v, psem).start()

    for b in range(_NBUF):
        pltpu.make_async_copy(
            table_ref.at[idx_v.at[b]], rows_v.at[b], gsems.at[b]
        ).start()

    @pl.loop(0, nchunk, step=_NBUF)
    def _chunks(c):
        for b in range(_NBUF):
            cc = c + b
            pltpu.make_async_copy(
                table_ref.at[idx_v.at[cc]], rows_v.at[b], gsems.at[b]
            ).wait()
            pltpu.sync_copy(
                rows_v.at[b], out_ref.at[pl.ds(base + cc * _K, _K)]
            )

            @pl.when(cc + _NBUF < nchunk)
            def _():
                pltpu.make_async_copy(
                    table_ref.at[idx_v.at[cc + _NBUF]], rows_v.at[b],
                    gsems.at[b]
                ).start()

    pltpu.make_async_copy(tflat_ref.at[pick_v], picked_v, psem).wait()
    acc = jnp.zeros((16,), jnp.float32)
    for c in range(bpw // 16):
        acc = acc + picked_v[pl.ds(c * 16, 16)]
    part_v[...] = acc
    pltpu.sync_copy(part_v, part_ref.at[w])


def _tc_logz_kernel(n_steps, idx_ref, table_ref, lsum_ref, rows, sems):
    i = pl.program_id(0)
    slot = lax.rem(i, 2)

    @pl.when(i == 0)
    def _prime():
        for j in range(_CH):
            pltpu.make_async_copy(
                table_ref.at[idx_ref[j]], rows.at[0, j], sems.at[0, j]
            ).start()

    @pl.when(i + 1 < n_steps)
    def _prefetch_next():
        nxt = lax.rem(i + 1, 2)
        for j in range(_CH):
            pltpu.make_async_copy(
                table_ref.at[idx_ref[(i + 1) * _CH + j]],
                rows.at[nxt, j],
                sems.at[nxt, j],
            ).start()

    for j in range(_CH):
        pltpu.make_async_copy(
            table_ref.at[idx_ref[i * _CH + j]], rows.at[slot, j],
            sems.at[slot, j]
        ).wait()

    block = rows[slot]
    m = jnp.max(block, axis=1, keepdims=True)
    s = jnp.sum(jnp.exp(block - m), axis=1, keepdims=True)
    part = jnp.sum(m + jnp.log(s))

    @pl.when(i == 0)
    def _init():
        lsum_ref[...] = jnp.zeros((1, 1), jnp.float32)

    lsum_ref[...] += part


def kernel(idx, targets, table):
    B, T = idx.shape
    V, C = table.shape
    n_tok = B * T
    idx_flat = idx.reshape(n_tok).astype(jnp.int32)
    tgt_flat = targets.reshape(n_tok).astype(jnp.int32)

    info = plsc.get_sparse_core_info()
    nc, ns = info.num_cores, info.num_subcores
    nw = nc * ns
    bpw = n_tok // nw
    nchunk = bpw // _K

    idx3d = idx_flat.reshape(nw, nchunk, _K)
    pick2d = (idx_flat * C + tgt_flat).reshape(nw, bpw)
    table_flat = table.reshape(V * C)

    sc_call = pl.kernel(
        functools.partial(_sc_gather_kernel, nc, bpw, nchunk),
        out_type=[
            jax.ShapeDtypeStruct((n_tok, C), jnp.float32),
            jax.ShapeDtypeStruct((nw, 16), jnp.float32),
        ],
        mesh=plsc.VectorSubcoreMesh(
            core_axis_name="c", subcore_axis_name="s"
        ),
        scratch_types=[
            pltpu.VMEM((nchunk, _K), jnp.int32),
            pltpu.VMEM((bpw,), jnp.int32),
            pltpu.VMEM((bpw,), jnp.float32),
            pltpu.VMEM((_NBUF, _K, C), jnp.float32),
            pltpu.VMEM((16,), jnp.float32),
            pltpu.SemaphoreType.DMA((_NBUF,)),
            pltpu.SemaphoreType.DMA,
        ],
    )
    logits_flat, partials = sc_call(table, table_flat, idx3d, pick2d)

    n_steps = n_tok // _CH
    grid_spec = pltpu.PrefetchScalarGridSpec(
        num_scalar_prefetch=1,
        grid=(n_steps,),
        in_specs=[
            pl.BlockSpec(memory_space=pltpu.HBM),
        ],
        out_specs=[
            pl.BlockSpec((1, 1), lambda i, idx_ref: (0, 0)),
        ],
        scratch_shapes=[
            pltpu.VMEM((2, _CH, C), jnp.float32),
            pltpu.SemaphoreType.DMA((2, _CH)),
        ],
    )
    lsum = pl.pallas_call(
        functools.partial(_tc_logz_kernel, n_steps),
        grid_spec=grid_spec,
        out_shape=[
            jax.ShapeDtypeStruct((1, 1), jnp.float32),
        ],
    )(idx_flat, table)[0]

    loss = (lsum[0, 0] - jnp.sum(partials)) / n_tok
    return logits_flat.reshape(B, T, C), loss


# speedup vs baseline: 2.7356x; 2.7356x over previous
import functools

import jax
import jax.numpy as jnp
from jax import lax
from jax.experimental import pallas as pl
from jax.experimental.pallas import tpu as pltpu
from jax.experimental.pallas import tpu_sc as plsc

_K = 4
_NBUF = 2
_BLK = 256


def _sc_gather_kernel(nc, bpw, nchunk, table_ref, idx_ref, out_ref, idx_v,
                      rows_v, gsems):
    w = lax.axis_index("s") * nc + lax.axis_index("c")
    base = w * bpw
    pltpu.sync_copy(idx_ref.at[w], idx_v)

    for b in range(_NBUF):
        pltpu.make_async_copy(
            table_ref.at[idx_v.at[b]], rows_v.at[b], gsems.at[b]
        ).start()

    @pl.loop(0, nchunk, step=_NBUF)
    def _chunks(c):
        for b in range(_NBUF):
            cc = c + b
            pltpu.make_async_copy(
                table_ref.at[idx_v.at[cc]], rows_v.at[b], gsems.at[b]
            ).wait()
            pltpu.sync_copy(
                rows_v.at[b], out_ref.at[pl.ds(base + cc * _K, _K)]
            )

            @pl.when(cc + _NBUF < nchunk)
            def _():
                pltpu.make_async_copy(
                    table_ref.at[idx_v.at[cc + _NBUF]], rows_v.at[b],
                    gsems.at[b]
                ).start()


def _tc_loss_kernel(logits_ref, tgt_ref, acc_ref):
    i = pl.program_id(0)
    block = logits_ref[...]
    m = jnp.max(block, axis=1, keepdims=True)
    s = jnp.sum(jnp.exp(block - m), axis=1, keepdims=True)
    logz = m + jnp.log(s)
    lanes = lax.broadcasted_iota(jnp.int32, block.shape, 1)
    onehot = lanes == tgt_ref[...]
    picked = jnp.sum(jnp.where(onehot, block, 0.0), axis=1, keepdims=True)
    part = jnp.sum(logz - picked)

    @pl.when(i == 0)
    def _init():
        acc_ref[...] = jnp.zeros((1, 1), jnp.float32)

    acc_ref[...] += part


def kernel(idx, targets, table):
    B, T = idx.shape
    V, C = table.shape
    n_tok = B * T
    idx_flat = idx.reshape(n_tok).astype(jnp.int32)
    tgt_flat = targets.reshape(n_tok).astype(jnp.int32)

    info = plsc.get_sparse_core_info()
    nc, ns = info.num_cores, info.num_subcores
    nw = nc * ns
    bpw = n_tok // nw
    nchunk = bpw // _K

    idx3d = idx_flat.reshape(nw, nchunk, _K)

    sc_call = pl.kernel(
        functools.partial(_sc_gather_kernel, nc, bpw, nchunk),
        out_type=jax.ShapeDtypeStruct((n_tok, C), jnp.float32),
        mesh=plsc.VectorSubcoreMesh(
            core_axis_name="c", subcore_axis_name="s"
        ),
        scratch_types=[
            pltpu.VMEM((nchunk, _K), jnp.int32),
            pltpu.VMEM((_NBUF, _K, C), jnp.float32),
            pltpu.SemaphoreType.DMA((_NBUF,)),
        ],
    )
    logits_flat = sc_call(table, idx3d)

    loss_sum = pl.pallas_call(
        _tc_loss_kernel,
        grid=(n_tok // _BLK,),
        in_specs=[
            pl.BlockSpec((_BLK, C), lambda i: (i, 0)),
            pl.BlockSpec((_BLK, 1), lambda i: (i, 0)),
        ],
        out_specs=pl.BlockSpec((1, 1), lambda i: (0, 0)),
        out_shape=jax.ShapeDtypeStruct((1, 1), jnp.float32),
    )(logits_flat, tgt_flat.reshape(n_tok, 1))

    loss = loss_sum[0, 0] / n_tok
    return logits_flat.reshape(B, T, C), loss


# speedup vs baseline: 2.7951x; 1.0218x over previous
import functools

import jax
import jax.numpy as jnp
from jax import lax
from jax.experimental import pallas as pl
from jax.experimental.pallas import tpu as pltpu
from jax.experimental.pallas import tpu_sc as plsc

_K = 4
_NBUF = 2
_BLK = 512


def _sc_gather_kernel(nc, bpw, nchunk, table_ref, idx_ref, out_ref, idx_v,
                      rows_v, gsems):
    w = lax.axis_index("s") * nc + lax.axis_index("c")
    base = w * bpw
    pltpu.sync_copy(idx_ref.at[w], idx_v)

    for b in range(_NBUF):
        pltpu.make_async_copy(
            table_ref.at[idx_v.at[b]], rows_v.at[b], gsems.at[b]
        ).start()

    @pl.loop(0, nchunk, step=_NBUF)
    def _chunks(c):
        for b in range(_NBUF):
            cc = c + b
            pltpu.make_async_copy(
                table_ref.at[idx_v.at[cc]], rows_v.at[b], gsems.at[b]
            ).wait()
            pltpu.sync_copy(
                rows_v.at[b], out_ref.at[pl.ds(base + cc * _K, _K)]
            )

            @pl.when(cc + _NBUF < nchunk)
            def _():
                pltpu.make_async_copy(
                    table_ref.at[idx_v.at[cc + _NBUF]], rows_v.at[b],
                    gsems.at[b]
                ).start()


def _tc_loss_kernel(logits_ref, tgt_ref, acc_ref):
    i = pl.program_id(0)
    block = logits_ref[...]
    m = jnp.max(block, axis=1, keepdims=True)
    s = jnp.sum(jnp.exp(block - m), axis=1, keepdims=True)
    logz = m + jnp.log(s)
    lanes = lax.broadcasted_iota(jnp.int32, block.shape, 1)
    onehot = lanes == tgt_ref[...]
    picked = jnp.sum(jnp.where(onehot, block, 0.0), axis=1, keepdims=True)
    part = jnp.sum(logz - picked)

    @pl.when(i == 0)
    def _init():
        acc_ref[...] = jnp.zeros((1, 1), jnp.float32)

    acc_ref[...] += part


def kernel(idx, targets, table):
    B, T = idx.shape
    V, C = table.shape
    n_tok = B * T
    idx_flat = idx.reshape(n_tok).astype(jnp.int32)
    tgt_flat = targets.reshape(n_tok).astype(jnp.int32)

    info = plsc.get_sparse_core_info()
    nc, ns = info.num_cores, info.num_subcores
    nw = nc * ns
    bpw = n_tok // nw
    nchunk = bpw // _K

    idx3d = idx_flat.reshape(nw, nchunk, _K)

    sc_call = pl.kernel(
        functools.partial(_sc_gather_kernel, nc, bpw, nchunk),
        out_type=jax.ShapeDtypeStruct((n_tok, C), jnp.float32),
        mesh=plsc.VectorSubcoreMesh(
            core_axis_name="c", subcore_axis_name="s"
        ),
        scratch_types=[
            pltpu.VMEM((nchunk, _K), jnp.int32),
            pltpu.VMEM((_NBUF, _K, C), jnp.float32),
            pltpu.SemaphoreType.DMA((_NBUF,)),
        ],
    )
    logits_flat = sc_call(table, idx3d)

    loss_sum = pl.pallas_call(
        _tc_loss_kernel,
        grid=(n_tok // _BLK,),
        in_specs=[
            pl.BlockSpec((_BLK, C), lambda i: (i, 0)),
            pl.BlockSpec((_BLK, 1), lambda i: (i, 0)),
        ],
        out_specs=pl.BlockSpec((1, 1), lambda i: (0, 0)),
        out_shape=jax.ShapeDtypeStruct((1, 1), jnp.float32),
    )(logits_flat, tgt_flat.reshape(n_tok, 1))

    loss = loss_sum[0, 0] / n_tok
    return logits_flat.reshape(B, T, C), loss
